# baseline (speedup 1.0000x reference)
import functools

import jax
import jax.numpy as jnp
from jax import lax
from jax.experimental import pallas as pl
from jax.experimental.pallas import tpu as pltpu
from jax.experimental.pallas import tpu_sc as plsc

NUM_EMBEDDINGS = 1000000
EMBEDDING_DIM = 64
BATCH = 16384
FIELDS = 26

B_TOTAL = BATCH * FIELDS
NW = 32
B_PER_W = B_TOTAL // NW
CHUNK = 832
N_CHUNKS = B_PER_W // CHUNK

PAIR_SPLIT = 512000
PAIR_ROWS = PAIR_SPLIT
T_ROWS = 512
T_GRID = PAIR_ROWS // T_ROWS
_LAST_B = (NUM_EMBEDDINGS - 1) // T_ROWS


def _transpose_body(a_ref, b_ref, o_ref):
    eye = (lax.broadcasted_iota(jnp.int32, (EMBEDDING_DIM, EMBEDDING_DIM), 0)
           == lax.broadcasted_iota(jnp.int32, (EMBEDDING_DIM, EMBEDDING_DIM), 1)
           ).astype(jnp.float32)
    dims = (((0,), (0,)), ((), ()))
    o_ref[:, 0:EMBEDDING_DIM] = lax.dot_general(
        a_ref[...], eye, dims, precision=lax.Precision.HIGHEST)
    o_ref[:, EMBEDDING_DIM:2 * EMBEDDING_DIM] = lax.dot_general(
        b_ref[...], eye, dims, precision=lax.Precision.HIGHEST)


_transpose_call = pl.pallas_call(
    _transpose_body,
    grid=(T_GRID,),
    in_specs=[
        pl.BlockSpec((EMBEDDING_DIM, T_ROWS), lambda i: (0, i)),
        pl.BlockSpec((EMBEDDING_DIM, T_ROWS),
                     lambda i: (0, jnp.minimum(T_GRID + i, _LAST_B))),
    ],
    out_specs=pl.BlockSpec((T_ROWS, 2 * EMBEDDING_DIM), lambda i: (i, 0)),
    out_shape=jax.ShapeDtypeStruct((PAIR_ROWS, 2 * EMBEDDING_DIM), jnp.float32),
)


def _make_gather_kernel():
    mesh = plsc.VectorSubcoreMesh(core_axis_name="c", subcore_axis_name="s")

    @functools.partial(
        pl.kernel,
        mesh=mesh,
        out_type=jax.ShapeDtypeStruct((B_TOTAL, EMBEDDING_DIM), jnp.float32),
        compiler_params=pltpu.CompilerParams(use_tc_tiling_on_sc=False),
        scratch_types=[
            pltpu.VMEM((CHUNK,), jnp.int32),
            pltpu.VMEM((CHUNK,), jnp.int32),
            pltpu.VMEM((CHUNK, EMBEDDING_DIM), jnp.float32),
            pltpu.VMEM((CHUNK, EMBEDDING_DIM), jnp.float32),
            pltpu.SemaphoreType.DMA,
            pltpu.SemaphoreType.DMA,
            pltpu.SemaphoreType.DMA,
            pltpu.SemaphoreType.DMA,
        ],
    )
    def gather_kernel(table_hbm, idx_hbm, out_hbm,
                      idx0, idx1, rows0, rows1,
                      gsem0, gsem1, wsem0, wsem1):
        wid = lax.axis_index("s") * 2 + lax.axis_index("c")
        w_base = wid * B_PER_W

        idx_v = (idx0, idx1)
        rows_v = (rows0, rows1)
        gsem = (gsem0, gsem1)
        wsem = (wsem0, wsem1)
        gathers = [None, None]
        writes = [None, None]

        for i in range(N_CHUNKS):
            b = i % 2
            base = w_base + i * CHUNK
            if writes[b] is not None:
                writes[b].wait()
            pltpu.sync_copy(idx_hbm.at[pl.ds(base, CHUNK)], idx_v[b])

            def remap(j, carry, _ref=idx_v[b]):
                v = _ref[pl.ds(j * 16, 16)]
                v2 = jnp.where(v < PAIR_SPLIT, v + v,
                               v + v - (2 * PAIR_SPLIT - 1))
                _ref[pl.ds(j * 16, 16)] = v2
                return carry

            lax.fori_loop(0, CHUNK // 16, remap, 0)
            gathers[b] = pltpu.async_copy(table_hbm.at[idx_v[b]], rows_v[b], gsem[b])
            if i >= 1:
                pb = (i - 1) % 2
                pbase = w_base + (i - 1) * CHUNK
                gathers[pb].wait()
                writes[pb] = pltpu.async_copy(
                    rows_v[pb], out_hbm.at[pl.ds(pbase, CHUNK)], wsem[pb])

        last = N_CHUNKS - 1
        lb = last % 2
        gathers[lb].wait()
        writes[lb] = pltpu.async_copy(
            rows_v[lb], out_hbm.at[pl.ds(w_base + last * CHUNK, CHUNK)], wsem[lb])
        writes[0].wait()
        writes[1].wait()

    return gather_kernel


_gather = _make_gather_kernel()


@jax.jit
def kernel(input_indices, weight):
    wT = weight.T
    pairs = _transpose_call(wT, wT)
    table = pairs.reshape(2 * PAIR_SPLIT, EMBEDDING_DIM)
    idx_flat = input_indices.reshape(B_TOTAL)
    out_flat = _gather(table, idx_flat)
    return out_flat.reshape(BATCH, FIELDS, EMBEDDING_DIM)


# speedup vs baseline: 1.8005x; 1.8005x over previous
import functools

import jax
import jax.numpy as jnp
from jax import lax
from jax.experimental import pallas as pl
from jax.experimental.pallas import tpu as pltpu
from jax.experimental.pallas import tpu_sc as plsc

NUM_EMBEDDINGS = 1000000
EMBEDDING_DIM = 64
BATCH = 16384
FIELDS = 26

B_TOTAL = BATCH * FIELDS
NW = 32
B_PER_W = B_TOTAL // NW
CHUNK = 832
N_CHUNKS = B_PER_W // CHUNK

PAIR_SPLIT = 512000
T_ROWS = 2048
T_GRID = PAIR_SPLIT // T_ROWS
_LAST_B = (NUM_EMBEDDINGS - 1) // T_ROWS


def _transpose_body(a_ref, b_ref, o_ref):
    o_ref[:, 0:EMBEDDING_DIM] = a_ref[...].T
    o_ref[:, EMBEDDING_DIM:2 * EMBEDDING_DIM] = b_ref[...].T


_transpose_call = pl.pallas_call(
    _transpose_body,
    grid=(T_GRID,),
    in_specs=[
        pl.BlockSpec((EMBEDDING_DIM, T_ROWS), lambda i: (0, i)),
        pl.BlockSpec((EMBEDDING_DIM, T_ROWS),
                     lambda i: (0, jnp.minimum(T_GRID + i, _LAST_B))),
    ],
    out_specs=pl.BlockSpec((T_ROWS, 2 * EMBEDDING_DIM), lambda i: (i, 0)),
    out_shape=jax.ShapeDtypeStruct((PAIR_SPLIT, 2 * EMBEDDING_DIM), jnp.float32),
)


def _make_gather_kernel():
    mesh = plsc.VectorSubcoreMesh(core_axis_name="c", subcore_axis_name="s")

    @functools.partial(
        pl.kernel,
        mesh=mesh,
        out_type=jax.ShapeDtypeStruct((B_TOTAL, EMBEDDING_DIM), jnp.float32),
        compiler_params=pltpu.CompilerParams(use_tc_tiling_on_sc=False),
        scratch_types=[
            pltpu.VMEM((CHUNK,), jnp.int32),
            pltpu.VMEM((CHUNK,), jnp.int32),
            pltpu.VMEM((CHUNK, EMBEDDING_DIM), jnp.float32),
            pltpu.VMEM((CHUNK, EMBEDDING_DIM), jnp.float32),
            pltpu.SemaphoreType.DMA,
            pltpu.SemaphoreType.DMA,
            pltpu.SemaphoreType.DMA,
            pltpu.SemaphoreType.DMA,
        ],
    )
    def gather_kernel(table_hbm, idx_hbm, out_hbm,
                      idx0, idx1, rows0, rows1,
                      gsem0, gsem1, wsem0, wsem1):
        wid = lax.axis_index("s") * 2 + lax.axis_index("c")
        w_base = wid * B_PER_W

        idx_v = (idx0, idx1)
        rows_v = (rows0, rows1)
        gsem = (gsem0, gsem1)
        wsem = (wsem0, wsem1)
        gathers = [None, None]
        writes = [None, None]

        for i in range(N_CHUNKS):
            b = i % 2
            base = w_base + i * CHUNK
            if writes[b] is not None:
                writes[b].wait()
            pltpu.sync_copy(idx_hbm.at[pl.ds(base, CHUNK)], idx_v[b])

            def remap(j, carry, _ref=idx_v[b]):
                v = _ref[pl.ds(j * 16, 16)]
                v2 = jnp.where(v < PAIR_SPLIT, v + v,
                               v + v - (2 * PAIR_SPLIT - 1))
                _ref[pl.ds(j * 16, 16)] = v2
                return carry

            lax.fori_loop(0, CHUNK // 16, remap, 0)
            gathers[b] = pltpu.async_copy(table_hbm.at[idx_v[b]], rows_v[b], gsem[b])
            if i >= 1:
                pb = (i - 1) % 2
                pbase = w_base + (i - 1) * CHUNK
                gathers[pb].wait()
                writes[pb] = pltpu.async_copy(
                    rows_v[pb], out_hbm.at[pl.ds(pbase, CHUNK)], wsem[pb])

        last = N_CHUNKS - 1
        lb = last % 2
        gathers[lb].wait()
        writes[lb] = pltpu.async_copy(
            rows_v[lb], out_hbm.at[pl.ds(w_base + last * CHUNK, CHUNK)], wsem[lb])
        writes[0].wait()
        writes[1].wait()

    return gather_kernel


_gather = _make_gather_kernel()


@jax.jit
def kernel(input_indices, weight):
    wT = weight.T
    pairs = _transpose_call(wT, wT)
    table = pairs.reshape(2 * PAIR_SPLIT, EMBEDDING_DIM)
    idx_flat = input_indices.reshape(B_TOTAL)
    out_flat = _gather(table, idx_flat)
    return out_flat.reshape(BATCH, FIELDS, EMBEDDING_DIM)


# speedup vs baseline: 1.9929x; 1.1069x over previous
import functools

import jax
import jax.numpy as jnp
from jax import lax
from jax.experimental import pallas as pl
from jax.experimental.pallas import tpu as pltpu
from jax.experimental.pallas import tpu_sc as plsc

NUM_EMBEDDINGS = 1000000
EMBEDDING_DIM = 64
BATCH = 16384
FIELDS = 26

B_TOTAL = BATCH * FIELDS
NW = 32
B_PER_W = B_TOTAL // NW
CHUNK = 832
N_CHUNKS = B_PER_W // CHUNK

PAIR_SPLIT = 512000
T_ROWS = 4096
T_GRID = PAIR_SPLIT // T_ROWS
_LAST_B = (NUM_EMBEDDINGS - 1) // T_ROWS


def _transpose_body(a_ref, b_ref, o_ref):
    o_ref[:, 0:EMBEDDING_DIM] = a_ref[...].T
    o_ref[:, EMBEDDING_DIM:2 * EMBEDDING_DIM] = b_ref[...].T


_transpose_call = pl.pallas_call(
    _transpose_body,
    grid=(T_GRID,),
    in_specs=[
        pl.BlockSpec((EMBEDDING_DIM, T_ROWS), lambda i: (0, i)),
        pl.BlockSpec((EMBEDDING_DIM, T_ROWS),
                     lambda i: (0, jnp.minimum(T_GRID + i, _LAST_B))),
    ],
    out_specs=pl.BlockSpec((T_ROWS, 2 * EMBEDDING_DIM), lambda i: (i, 0)),
    out_shape=jax.ShapeDtypeStruct((PAIR_SPLIT, 2 * EMBEDDING_DIM), jnp.float32),
)


def _make_gather_kernel():
    mesh = plsc.VectorSubcoreMesh(core_axis_name="c", subcore_axis_name="s")

    @functools.partial(
        pl.kernel,
        mesh=mesh,
        out_type=jax.ShapeDtypeStruct((B_TOTAL, EMBEDDING_DIM), jnp.float32),
        compiler_params=pltpu.CompilerParams(use_tc_tiling_on_sc=False),
        scratch_types=[
            pltpu.VMEM((CHUNK,), jnp.int32),
            pltpu.VMEM((CHUNK,), jnp.int32),
            pltpu.VMEM((CHUNK, EMBEDDING_DIM), jnp.float32),
            pltpu.VMEM((CHUNK, EMBEDDING_DIM), jnp.float32),
            pltpu.SemaphoreType.DMA,
            pltpu.SemaphoreType.DMA,
            pltpu.SemaphoreType.DMA,
            pltpu.SemaphoreType.DMA,
        ],
    )
    def gather_kernel(table_hbm, idx_hbm, out_hbm,
                      idx0, idx1, rows0, rows1,
                      gsem0, gsem1, wsem0, wsem1):
        wid = lax.axis_index("s") * 2 + lax.axis_index("c")
        w_base = wid * B_PER_W

        idx_v = (idx0, idx1)
        rows_v = (rows0, rows1)
        gsem = (gsem0, gsem1)
        wsem = (wsem0, wsem1)
        gathers = [None, None]
        writes = [None, None]

        for i in range(N_CHUNKS):
            b = i % 2
            base = w_base + i * CHUNK
            if writes[b] is not None:
                writes[b].wait()
            pltpu.sync_copy(idx_hbm.at[pl.ds(base, CHUNK)], idx_v[b])

            def remap(j, carry, _ref=idx_v[b]):
                v = _ref[pl.ds(j * 16, 16)]
                v2 = jnp.where(v < PAIR_SPLIT, v + v,
                               v + v - (2 * PAIR_SPLIT - 1))
                _ref[pl.ds(j * 16, 16)] = v2
                return carry

            lax.fori_loop(0, CHUNK // 16, remap, 0)
            gathers[b] = pltpu.async_copy(table_hbm.at[idx_v[b]], rows_v[b], gsem[b])
            if i >= 1:
                pb = (i - 1) % 2
                pbase = w_base + (i - 1) * CHUNK
                gathers[pb].wait()
                writes[pb] = pltpu.async_copy(
                    rows_v[pb], out_hbm.at[pl.ds(pbase, CHUNK)], wsem[pb])

        last = N_CHUNKS - 1
        lb = last % 2
        gathers[lb].wait()
        writes[lb] = pltpu.async_copy(
            rows_v[lb], out_hbm.at[pl.ds(w_base + last * CHUNK, CHUNK)], wsem[lb])
        writes[0].wait()
        writes[1].wait()

    return gather_kernel


_gather = _make_gather_kernel()


@jax.jit
def kernel(input_indices, weight):
    wT = weight.T
    pairs = _transpose_call(wT, wT)
    table = pairs.reshape(2 * PAIR_SPLIT, EMBEDDING_DIM)
    idx_flat = input_indices.reshape(B_TOTAL)
    out_flat = _gather(table, idx_flat)
    return out_flat.reshape(BATCH, FIELDS, EMBEDDING_DIM)


# speedup vs baseline: 2.0699x; 1.0386x over previous
import functools

import jax
import jax.numpy as jnp
from jax import lax
from jax.experimental import pallas as pl
from jax.experimental.pallas import tpu as pltpu
from jax.experimental.pallas import tpu_sc as plsc

NUM_EMBEDDINGS = 1000000
EMBEDDING_DIM = 64
BATCH = 16384
FIELDS = 26

B_TOTAL = BATCH * FIELDS
NW = 32
B_PER_W = B_TOTAL // NW
CHUNK = 832
N_CHUNKS = B_PER_W // CHUNK

PAIR_SPLIT = 512000
T_ROWS = 6400
T_GRID = PAIR_SPLIT // T_ROWS
_LAST_B = (NUM_EMBEDDINGS - 1) // T_ROWS


def _transpose_body(a_ref, b_ref, o_ref):
    o_ref[:, 0:EMBEDDING_DIM] = a_ref[...].T
    o_ref[:, EMBEDDING_DIM:2 * EMBEDDING_DIM] = b_ref[...].T


_transpose_call = pl.pallas_call(
    _transpose_body,
    grid=(T_GRID,),
    in_specs=[
        pl.BlockSpec((EMBEDDING_DIM, T_ROWS), lambda i: (0, i)),
        pl.BlockSpec((EMBEDDING_DIM, T_ROWS),
                     lambda i: (0, jnp.minimum(T_GRID + i, _LAST_B))),
    ],
    out_specs=pl.BlockSpec((T_ROWS, 2 * EMBEDDING_DIM), lambda i: (i, 0)),
    out_shape=jax.ShapeDtypeStruct((PAIR_SPLIT, 2 * EMBEDDING_DIM), jnp.float32),
)


def _make_gather_kernel():
    mesh = plsc.VectorSubcoreMesh(core_axis_name="c", subcore_axis_name="s")

    @functools.partial(
        pl.kernel,
        mesh=mesh,
        out_type=jax.ShapeDtypeStruct((B_TOTAL, EMBEDDING_DIM), jnp.float32),
        compiler_params=pltpu.CompilerParams(use_tc_tiling_on_sc=False),
        scratch_types=[
            pltpu.VMEM((CHUNK,), jnp.int32),
            pltpu.VMEM((CHUNK,), jnp.int32),
            pltpu.VMEM((CHUNK, EMBEDDING_DIM), jnp.float32),
            pltpu.VMEM((CHUNK, EMBEDDING_DIM), jnp.float32),
            pltpu.SemaphoreType.DMA,
            pltpu.SemaphoreType.DMA,
            pltpu.SemaphoreType.DMA,
            pltpu.SemaphoreType.DMA,
        ],
    )
    def gather_kernel(table_hbm, idx_hbm, out_hbm,
                      idx0, idx1, rows0, rows1,
                      gsem0, gsem1, wsem0, wsem1):
        wid = lax.axis_index("s") * 2 + lax.axis_index("c")
        w_base = wid * B_PER_W

        idx_v = (idx0, idx1)
        rows_v = (rows0, rows1)
        gsem = (gsem0, gsem1)
        wsem = (wsem0, wsem1)
        gathers = [None, None]
        writes = [None, None]

        for i in range(N_CHUNKS):
            b = i % 2
            base = w_base + i * CHUNK
            if writes[b] is not None:
                writes[b].wait()
            pltpu.sync_copy(idx_hbm.at[pl.ds(base, CHUNK)], idx_v[b])

            def remap(j, carry, _ref=idx_v[b]):
                v = _ref[pl.ds(j * 16, 16)]
                v2 = jnp.where(v < PAIR_SPLIT, v + v,
                               v + v - (2 * PAIR_SPLIT - 1))
                _ref[pl.ds(j * 16, 16)] = v2
                return carry

            lax.fori_loop(0, CHUNK // 16, remap, 0)
            gathers[b] = pltpu.async_copy(table_hbm.at[idx_v[b]], rows_v[b], gsem[b])
            if i >= 1:
                pb = (i - 1) % 2
                pbase = w_base + (i - 1) * CHUNK
                gathers[pb].wait()
                writes[pb] = pltpu.async_copy(
                    rows_v[pb], out_hbm.at[pl.ds(pbase, CHUNK)], wsem[pb])

        last = N_CHUNKS - 1
        lb = last % 2
        gathers[lb].wait()
        writes[lb] = pltpu.async_copy(
            rows_v[lb], out_hbm.at[pl.ds(w_base + last * CHUNK, CHUNK)], wsem[lb])
        writes[0].wait()
        writes[1].wait()

    return gather_kernel


_gather = _make_gather_kernel()


@jax.jit
def kernel(input_indices, weight):
    wT = weight.T
    pairs = _transpose_call(wT, wT)
    table = pairs.reshape(2 * PAIR_SPLIT, EMBEDDING_DIM)
    idx_flat = input_indices.reshape(B_TOTAL)
    out_flat = _gather(table, idx_flat)
    return out_flat.reshape(BATCH, FIELDS, EMBEDDING_DIM)
